# baseline (speedup 1.0000x reference)
import math

import jax
import jax.numpy as jnp
from jax.experimental import pallas as pl
from jax.experimental.pallas import tpu as pltpu

EPS = 1e-5
EPS_VAR = 1e-4
SQRT_2_OVER_PI = math.sqrt(2.0 / math.pi)

TC = 2048
TH = 128

_F32 = jnp.float32
_BF16 = jnp.bfloat16


def _body(sc_ref, vec_ref, rhs_ref, tril_ref, x_ref, o_ref, cum_x_ref, cum_sq_ref):
    j = pl.program_id(1)

    @pl.when(j == 0)
    def _():
        cum_x_ref[...] = jnp.zeros_like(cum_x_ref)
        cum_sq_ref[...] = jnp.zeros_like(cum_sq_ref)

    tau = sc_ref[0]
    sig1 = sc_ref[1]
    sig2 = sc_ref[2]
    sig3 = sc_ref[3]
    w = sc_ref[4]
    a1 = sc_ref[5]
    a2 = sc_ref[6]
    a3 = sc_ref[7]

    xb = x_ref[0]
    D = xb.shape[-1]
    xb16 = xb.astype(_BF16)
    x2_16 = xb16 * xb16
    tril16 = tril_ref[...]
    rhs16 = rhs_ref[...]

    inner16 = _BF16(SQRT_2_OVER_PI) * (xb16 + _BF16(0.044715) * (x2_16 * xb16))
    t = jnp.tanh(inner16.astype(_F32))
    out = xb * (0.5 * t + 0.5)

    m = vec_ref[0:1, :]
    sq = vec_ref[1:2, :]
    var_g = jnp.maximum(sq - m * m, EPS_VAR)
    inv1_16 = (1.0 / (jnp.sqrt(var_g) + EPS)).astype(_BF16)
    m16 = m.astype(_BF16)
    z1a = jnp.abs((xb16 - m16) * inv1_16)

    vf = vec_ref[3:4, :]
    vs = vec_ref[4:5, :]
    ratio = jnp.minimum(vf / jnp.maximum(vs, EPS_VAR), 10.0)
    burst = jnp.maximum(jnp.sum(ratio, axis=-1, keepdims=True) / D - 1.0, 0.0)
    surp2 = jnp.tanh(sig2 * burst)
    s2a = jnp.exp(a2 * jnp.log(jnp.maximum(surp2, 1e-7)))
    weff = w * s2a

    z3a_halves = []
    carry_x = cum_x_ref[...]
    carry_sq = cum_sq_ref[...]
    for h in range(TC // TH):
        lo = h * TH
        xh = xb16[lo:lo + TH, :]
        x2h = x2_16[lo:lo + TH, :]
        pre_x = carry_x + jax.lax.dot(tril16, xh, preferred_element_type=_F32)
        pre_sq = carry_sq + jax.lax.dot(tril16, x2h, preferred_element_type=_F32)
        xlast = xb[lo + TH - 1:lo + TH, :]
        carry_x = pre_x[TH - 1:TH, :] + xlast
        carry_sq = pre_sq[TH - 1:TH, :] + xlast * xlast

        t_row = j * TC + lo + jax.lax.broadcasted_iota(jnp.int32, (TH, 1), 0)
        cnt = jnp.maximum(t_row, 1).astype(_F32)
        cnt16 = cnt.astype(_BF16)
        evar16 = (EPS_VAR * cnt * cnt).astype(_BF16)
        pre16 = pre_x.astype(_BF16)
        psq16 = pre_sq.astype(_BF16)
        num16 = cnt16 * xh - pre16
        den2 = jnp.maximum(cnt16 * psq16 - pre16 * pre16, evar16)
        rden = jax.lax.rsqrt(den2.astype(_F32))
        z3a_halves.append(jnp.abs(num16 * rden.astype(_BF16)))
    cum_x_ref[...] = carry_x
    cum_sq_ref[...] = carry_sq
    z3a = jnp.concatenate(z3a_halves, axis=0)
    t_row = j * TC + jax.lax.broadcasted_iota(jnp.int32, (TC, 1), 0)

    out16 = out.astype(_BF16)
    outsq16 = out16 * out16
    r1 = jax.lax.dot(z1a, rhs16, preferred_element_type=_F32)
    r3 = jax.lax.dot(z3a, rhs16, preferred_element_type=_F32)
    rs = jax.lax.dot(outsq16, rhs16, preferred_element_type=_F32)
    rd = jax.lax.dot(out16, rhs16, preferred_element_type=_F32)
    sum1 = r1[:, 0:1]
    sum3 = r3[:, 0:1]
    sumsq = rs[:, 0:1]
    dote = rd[:, 1:2]

    surp1 = jnp.tanh((sig1 / D) * sum1)
    rowmask = (t_row > 0).astype(_F32)
    surp3 = jnp.tanh((sig3 / D) * (sum3 * rowmask))

    s1a = jnp.exp(a1 * jnp.log(jnp.maximum(surp1, 1e-7)))
    s3a = jnp.exp(a3 * jnp.log(jnp.maximum(surp3, 1e-7)))
    joint = s1a * s3a

    en = vec_ref[2:3, :]
    inv_en = jax.lax.rsqrt(jnp.maximum(jnp.sum(en * en, axis=-1, keepdims=True), 1e-24))
    inv_norm = jax.lax.rsqrt(jnp.maximum(sumsq, 1e-24))
    cos = jnp.clip(dote * (inv_en * inv_norm), -1.0, 1.0)
    gate_cos = jnp.exp(-tau * cos)

    gate = gate_cos * (1.0 + weff * joint)
    o_ref[0] = out * gate


def kernel(x, ema_mean, ema_sq, ema_out, var_fast, var_slow,
           log_tau, log_sig1, log_sig2, log_sig3, log_w_raw,
           log_a1, log_a2, log_a3):
    B, T, D = x.shape
    sp = jax.nn.softplus
    scalars = jnp.stack([
        jnp.exp(log_tau), sp(log_sig1), sp(log_sig2), sp(log_sig3),
        sp(log_w_raw), sp(log_a1), sp(log_a2), sp(log_a3),
    ]).astype(_F32)
    vecs = jnp.stack([ema_mean, ema_sq, ema_out, var_fast, var_slow], axis=0)
    rhs = jnp.stack([jnp.ones((D,), _F32), ema_out], axis=1).astype(_BF16)
    ii = jnp.arange(TH, dtype=jnp.int32)
    tril = (ii[None, :] < ii[:, None]).astype(_BF16)

    return pl.pallas_call(
        _body,
        grid=(B, T // TC),
        in_specs=[
            pl.BlockSpec(memory_space=pltpu.SMEM),
            pl.BlockSpec((5, D), lambda b, j: (0, 0)),
            pl.BlockSpec((D, 2), lambda b, j: (0, 0)),
            pl.BlockSpec((TH, TH), lambda b, j: (0, 0)),
            pl.BlockSpec((1, TC, D), lambda b, j: (b, j, 0)),
        ],
        out_specs=pl.BlockSpec((1, TC, D), lambda b, j: (b, j, 0)),
        out_shape=jax.ShapeDtypeStruct((B, T, D), _F32),
        scratch_shapes=[
            pltpu.VMEM((1, D), _F32),
            pltpu.VMEM((1, D), _F32),
        ],
        compiler_params=pltpu.CompilerParams(
            dimension_semantics=("parallel", "arbitrary"),
            vmem_limit_bytes=100 * 1024 * 1024,
        ),
    )(scalars, vecs, rhs, tril, x)


# speedup vs baseline: 1.0467x; 1.0467x over previous
import math

import jax
import jax.numpy as jnp
from jax.experimental import pallas as pl
from jax.experimental.pallas import tpu as pltpu

EPS = 1e-5
EPS_VAR = 1e-4
SQRT_2_OVER_PI = math.sqrt(2.0 / math.pi)

TC = 1024
TH = 64

_F32 = jnp.float32
_BF16 = jnp.bfloat16


def _body(sc_ref, vec_ref, rhs_ref, tril_ref, x_ref, o_ref, cum_x_ref, cum_sq_ref):
    j = pl.program_id(1)

    @pl.when(j == 0)
    def _():
        cum_x_ref[...] = jnp.zeros_like(cum_x_ref)
        cum_sq_ref[...] = jnp.zeros_like(cum_sq_ref)

    tau = sc_ref[0]
    sig1 = sc_ref[1]
    sig2 = sc_ref[2]
    sig3 = sc_ref[3]
    w = sc_ref[4]
    a1 = sc_ref[5]
    a2 = sc_ref[6]
    a3 = sc_ref[7]

    xb = x_ref[0]
    D = xb.shape[-1]
    xb16 = xb.astype(_BF16)
    x2_16 = xb16 * xb16
    tril16 = tril_ref[...]
    rhs16 = rhs_ref[...]

    inner16 = _BF16(SQRT_2_OVER_PI) * (xb16 + _BF16(0.044715) * (x2_16 * xb16))
    t = jnp.tanh(inner16.astype(_F32))
    out = xb * (0.5 * t + 0.5)

    m = vec_ref[0:1, :]
    sq = vec_ref[1:2, :]
    var_g = jnp.maximum(sq - m * m, EPS_VAR)
    inv1_16 = (1.0 / (jnp.sqrt(var_g) + EPS)).astype(_BF16)
    m16 = m.astype(_BF16)
    z1a = jnp.abs((xb16 - m16) * inv1_16)

    vf = vec_ref[3:4, :]
    vs = vec_ref[4:5, :]
    ratio = jnp.minimum(vf / jnp.maximum(vs, EPS_VAR), 10.0)
    burst = jnp.maximum(jnp.sum(ratio, axis=-1, keepdims=True) / D - 1.0, 0.0)
    surp2 = jnp.tanh(sig2 * burst)
    s2a = jnp.exp(a2 * jnp.log(jnp.maximum(surp2, 1e-7)))
    weff = w * s2a

    z3a_halves = []
    carry_x = cum_x_ref[...]
    carry_sq = cum_sq_ref[...]
    for h in range(TC // TH):
        lo = h * TH
        xh = xb16[lo:lo + TH, :]
        x2h = x2_16[lo:lo + TH, :]
        pre_x = carry_x + jax.lax.dot(tril16, xh, preferred_element_type=_F32)
        pre_sq = carry_sq + jax.lax.dot(tril16, x2h, preferred_element_type=_F32)
        xlast = xb[lo + TH - 1:lo + TH, :]
        carry_x = pre_x[TH - 1:TH, :] + xlast
        carry_sq = pre_sq[TH - 1:TH, :] + xlast * xlast

        t_row = j * TC + lo + jax.lax.broadcasted_iota(jnp.int32, (TH, 1), 0)
        cnt = jnp.maximum(t_row, 1).astype(_F32)
        cnt16 = cnt.astype(_BF16)
        evar16 = (EPS_VAR * cnt * cnt).astype(_BF16)
        pre16 = pre_x.astype(_BF16)
        psq16 = pre_sq.astype(_BF16)
        num16 = cnt16 * xh - pre16
        den2 = jnp.maximum(cnt16 * psq16 - pre16 * pre16, evar16)
        rden = jax.lax.rsqrt(den2.astype(_F32))
        z3a_halves.append(jnp.abs(num16 * rden.astype(_BF16)))
    cum_x_ref[...] = carry_x
    cum_sq_ref[...] = carry_sq
    z3a = jnp.concatenate(z3a_halves, axis=0)
    t_row = j * TC + jax.lax.broadcasted_iota(jnp.int32, (TC, 1), 0)

    out16 = out.astype(_BF16)
    outsq16 = out16 * out16
    r1 = jax.lax.dot(z1a, rhs16, preferred_element_type=_F32)
    r3 = jax.lax.dot(z3a, rhs16, preferred_element_type=_F32)
    rs = jax.lax.dot(outsq16, rhs16, preferred_element_type=_F32)
    rd = jax.lax.dot(out16, rhs16, preferred_element_type=_F32)
    sum1 = r1[:, 0:1]
    sum3 = r3[:, 0:1]
    sumsq = rs[:, 0:1]
    dote = rd[:, 1:2]

    surp1 = jnp.tanh((sig1 / D) * sum1)
    rowmask = (t_row > 0).astype(_F32)
    surp3 = jnp.tanh((sig3 / D) * (sum3 * rowmask))

    s1a = jnp.exp(a1 * jnp.log(jnp.maximum(surp1, 1e-7)))
    s3a = jnp.exp(a3 * jnp.log(jnp.maximum(surp3, 1e-7)))
    joint = s1a * s3a

    en = vec_ref[2:3, :]
    inv_en = jax.lax.rsqrt(jnp.maximum(jnp.sum(en * en, axis=-1, keepdims=True), 1e-24))
    inv_norm = jax.lax.rsqrt(jnp.maximum(sumsq, 1e-24))
    cos = jnp.clip(dote * (inv_en * inv_norm), -1.0, 1.0)
    gate_cos = jnp.exp(-tau * cos)

    gate = gate_cos * (1.0 + weff * joint)
    o_ref[0] = out * gate


def kernel(x, ema_mean, ema_sq, ema_out, var_fast, var_slow,
           log_tau, log_sig1, log_sig2, log_sig3, log_w_raw,
           log_a1, log_a2, log_a3):
    B, T, D = x.shape
    sp = jax.nn.softplus
    scalars = jnp.stack([
        jnp.exp(log_tau), sp(log_sig1), sp(log_sig2), sp(log_sig3),
        sp(log_w_raw), sp(log_a1), sp(log_a2), sp(log_a3),
    ]).astype(_F32)
    vecs = jnp.stack([ema_mean, ema_sq, ema_out, var_fast, var_slow], axis=0)
    rhs = jnp.stack([jnp.ones((D,), _F32), ema_out], axis=1).astype(_BF16)
    ii = jnp.arange(TH, dtype=jnp.int32)
    tril = (ii[None, :] < ii[:, None]).astype(_BF16)

    return pl.pallas_call(
        _body,
        grid=(B, T // TC),
        in_specs=[
            pl.BlockSpec(memory_space=pltpu.SMEM),
            pl.BlockSpec((5, D), lambda b, j: (0, 0)),
            pl.BlockSpec((D, 2), lambda b, j: (0, 0)),
            pl.BlockSpec((TH, TH), lambda b, j: (0, 0)),
            pl.BlockSpec((1, TC, D), lambda b, j: (b, j, 0)),
        ],
        out_specs=pl.BlockSpec((1, TC, D), lambda b, j: (b, j, 0)),
        out_shape=jax.ShapeDtypeStruct((B, T, D), _F32),
        scratch_shapes=[
            pltpu.VMEM((1, D), _F32),
            pltpu.VMEM((1, D), _F32),
        ],
        compiler_params=pltpu.CompilerParams(
            dimension_semantics=("parallel", "arbitrary"),
            vmem_limit_bytes=100 * 1024 * 1024,
        ),
    )(scalars, vecs, rhs, tril, x)


# speedup vs baseline: 1.0488x; 1.0020x over previous
import math

import jax
import jax.numpy as jnp
from jax.experimental import pallas as pl
from jax.experimental.pallas import tpu as pltpu

EPS = 1e-5
EPS_VAR = 1e-4
SQRT_2_OVER_PI = math.sqrt(2.0 / math.pi)

TC = 1024
TH = 64

_F32 = jnp.float32
_BF16 = jnp.bfloat16


def _body(sc_ref, vec_ref, rhs_ref, tril_ref, x_ref, o_ref, cum_x_ref, cum_sq_ref):
    j = pl.program_id(1)

    @pl.when(j == 0)
    def _():
        cum_x_ref[...] = jnp.zeros_like(cum_x_ref)
        cum_sq_ref[...] = jnp.zeros_like(cum_sq_ref)

    tau = sc_ref[0]
    sig1 = sc_ref[1]
    sig2 = sc_ref[2]
    sig3 = sc_ref[3]
    w = sc_ref[4]
    a1 = sc_ref[5]
    a2 = sc_ref[6]
    a3 = sc_ref[7]

    xb = x_ref[0]
    D = xb.shape[-1]
    xb16 = xb.astype(_BF16)
    x2_16 = xb16 * xb16
    tril16 = tril_ref[...]
    rhs16 = rhs_ref[...]

    inner16 = _BF16(SQRT_2_OVER_PI) * (xb16 + _BF16(0.044715) * (x2_16 * xb16))
    t = jnp.tanh(inner16.astype(_F32))
    out = xb * (0.5 * t + 0.5)

    m = vec_ref[0:1, :]
    sq = vec_ref[1:2, :]
    var_g = jnp.maximum(sq - m * m, EPS_VAR)
    inv1_16 = (1.0 / (jnp.sqrt(var_g) + EPS)).astype(_BF16)
    m16 = m.astype(_BF16)
    z1a = jnp.abs((xb16 - m16) * inv1_16)

    vf = vec_ref[3:4, :]
    vs = vec_ref[4:5, :]
    ratio = jnp.minimum(vf / jnp.maximum(vs, EPS_VAR), 10.0)
    burst = jnp.maximum(jnp.sum(ratio, axis=-1, keepdims=True) / D - 1.0, 0.0)
    surp2 = jnp.tanh(sig2 * burst)
    s2a = jnp.exp(a2 * jnp.log(jnp.maximum(surp2, 1e-7)))
    weff = w * s2a

    z3a_halves = []
    carry_x = cum_x_ref[...]
    carry_sq = cum_sq_ref[...]
    for h in range(TC // TH):
        lo = h * TH
        xh = xb16[lo:lo + TH, :]
        x2h = x2_16[lo:lo + TH, :]
        d_x = jax.lax.dot(tril16, xh, preferred_element_type=_F32)
        d_sq = jax.lax.dot(tril16, x2h, preferred_element_type=_F32)
        pre_x = carry_x + d_x
        pre_sq = carry_sq + d_sq
        xlast = xb[lo + TH - 1:lo + TH, :]
        carry_x = carry_x + (d_x[TH - 1:TH, :] + xlast)
        carry_sq = carry_sq + (d_sq[TH - 1:TH, :] + xlast * xlast)

        t_row = j * TC + lo + jax.lax.broadcasted_iota(jnp.int32, (TH, 1), 0)
        cnt = jnp.maximum(t_row, 1).astype(_F32)
        cnt16 = cnt.astype(_BF16)
        evar16 = (EPS_VAR * cnt * cnt).astype(_BF16)
        pre16 = pre_x.astype(_BF16)
        psq16 = pre_sq.astype(_BF16)
        num16 = cnt16 * xh - pre16
        den2 = jnp.maximum(cnt16 * psq16 - pre16 * pre16, evar16)
        rden = jax.lax.rsqrt(den2.astype(_F32))
        z3a_halves.append(jnp.abs(num16 * rden.astype(_BF16)))
    cum_x_ref[...] = carry_x
    cum_sq_ref[...] = carry_sq
    z3a = jnp.concatenate(z3a_halves, axis=0)
    t_row = j * TC + jax.lax.broadcasted_iota(jnp.int32, (TC, 1), 0)

    out16 = out.astype(_BF16)
    outsq16 = out16 * out16
    r1 = jax.lax.dot(z1a, rhs16, preferred_element_type=_F32)
    r3 = jax.lax.dot(z3a, rhs16, preferred_element_type=_F32)
    rs = jax.lax.dot(outsq16, rhs16, preferred_element_type=_F32)
    rd = jax.lax.dot(out16, rhs16, preferred_element_type=_F32)
    sum1 = r1[:, 0:1]
    sum3 = r3[:, 0:1]
    sumsq = rs[:, 0:1]
    dote = rd[:, 1:2]

    surp1 = jnp.tanh((sig1 / D) * sum1)
    rowmask = (t_row > 0).astype(_F32)
    surp3 = jnp.tanh((sig3 / D) * (sum3 * rowmask))

    s1a = jnp.exp(a1 * jnp.log(jnp.maximum(surp1, 1e-7)))
    s3a = jnp.exp(a3 * jnp.log(jnp.maximum(surp3, 1e-7)))
    joint = s1a * s3a

    en = vec_ref[2:3, :]
    inv_en = jax.lax.rsqrt(jnp.maximum(jnp.sum(en * en, axis=-1, keepdims=True), 1e-24))
    inv_norm = jax.lax.rsqrt(jnp.maximum(sumsq, 1e-24))
    cos = jnp.clip(dote * (inv_en * inv_norm), -1.0, 1.0)
    gate_cos = jnp.exp(-tau * cos)

    gate = gate_cos * (1.0 + weff * joint)
    o_ref[0] = out * gate


def kernel(x, ema_mean, ema_sq, ema_out, var_fast, var_slow,
           log_tau, log_sig1, log_sig2, log_sig3, log_w_raw,
           log_a1, log_a2, log_a3):
    B, T, D = x.shape
    sp = jax.nn.softplus
    scalars = jnp.stack([
        jnp.exp(log_tau), sp(log_sig1), sp(log_sig2), sp(log_sig3),
        sp(log_w_raw), sp(log_a1), sp(log_a2), sp(log_a3),
    ]).astype(_F32)
    vecs = jnp.stack([ema_mean, ema_sq, ema_out, var_fast, var_slow], axis=0)
    rhs = jnp.stack([jnp.ones((D,), _F32), ema_out], axis=1).astype(_BF16)
    ii = jnp.arange(TH, dtype=jnp.int32)
    tril = (ii[None, :] < ii[:, None]).astype(_BF16)

    return pl.pallas_call(
        _body,
        grid=(B, T // TC),
        in_specs=[
            pl.BlockSpec(memory_space=pltpu.SMEM),
            pl.BlockSpec((5, D), lambda b, j: (0, 0)),
            pl.BlockSpec((D, 2), lambda b, j: (0, 0)),
            pl.BlockSpec((TH, TH), lambda b, j: (0, 0)),
            pl.BlockSpec((1, TC, D), lambda b, j: (b, j, 0)),
        ],
        out_specs=pl.BlockSpec((1, TC, D), lambda b, j: (b, j, 0)),
        out_shape=jax.ShapeDtypeStruct((B, T, D), _F32),
        scratch_shapes=[
            pltpu.VMEM((1, D), _F32),
            pltpu.VMEM((1, D), _F32),
        ],
        compiler_params=pltpu.CompilerParams(
            dimension_semantics=("parallel", "arbitrary"),
            vmem_limit_bytes=100 * 1024 * 1024,
        ),
    )(scalars, vecs, rhs, tril, x)
